# Initial kernel scaffold; baseline (speedup 1.0000x reference)
import jax
import jax.numpy as jnp
from jax.experimental import pallas as pl


def kernel(x, mask):
    raise NotImplementedError("write your pallas kernel here")


# speedup vs baseline: 1.8849x; 1.8849x over previous
import functools

import jax
import jax.numpy as jnp
from jax.experimental import pallas as pl
from jax.experimental.pallas import tpu as pltpu

SB = 256
D = 1024
S = 8192
B = 4


def _body(x_ref, m_ref, o_ref, carry_ref):
    j = pl.program_id(1)

    @pl.when(j == 0)
    def _():
        carry_ref[...] = jnp.zeros_like(carry_ref)

    xm = jnp.where(m_ref[0], x_ref[0], 0.0)
    row = jax.lax.broadcasted_iota(jnp.int32, (SB, SB), 0)
    col = jax.lax.broadcasted_iota(jnp.int32, (SB, SB), 1)
    tri = (row >= col).astype(jnp.float32)
    acc = jax.lax.dot(tri, xm, preferred_element_type=jnp.float32)
    out = acc + carry_ref[...]
    o_ref[...] = out[None]
    carry_ref[...] = out[-1:, :]


@jax.jit
def kernel(x, mask):
    grid = (B, S // SB)
    return pl.pallas_call(
        _body,
        grid=grid,
        in_specs=[
            pl.BlockSpec((1, SB, D), lambda b, j: (b, j, 0)),
            pl.BlockSpec((1, SB, D), lambda b, j: (b, j, 0)),
        ],
        out_specs=pl.BlockSpec((1, SB, D), lambda b, j: (b, j, 0)),
        out_shape=jax.ShapeDtypeStruct((B, S, D), jnp.float32),
        scratch_shapes=[pltpu.VMEM((1, D), jnp.float32)],
        compiler_params=pltpu.CompilerParams(
            dimension_semantics=("arbitrary", "arbitrary"),
        ),
    )(x, mask)
